# baseline (speedup 1.0000x reference)
import functools

import jax
import jax.numpy as jnp
from jax import lax
from jax.experimental import pallas as pl
from jax.experimental.pallas import tpu as pltpu
from jax.experimental.pallas import tpu_sc as plsc

BATCH = 4
SEQ = 2048
DIM = 2048
NW = 32
S_PER_W = SEQ // NW
ROWS = 16
CHUNK = ROWS * DIM
N_CHUNK = S_PER_W // ROWS
LANES = 16


def _make_sc_kernel():
    mesh = plsc.VectorSubcoreMesh(core_axis_name="c", subcore_axis_name="s")

    @functools.partial(
        pl.kernel,
        mesh=mesh,
        out_type=jax.ShapeDtypeStruct((BATCH * SEQ * DIM,), jnp.float32),
        scratch_types=[
            pltpu.VMEM((CHUNK,), jnp.float32),
            pltpu.VMEM((CHUNK,), jnp.float32),
            pltpu.VMEM((CHUNK,), jnp.float32),
        ],
    )
    def sc_add(x_hbm, pe_hbm, out_hbm, x_buf, pe_buf, o_buf):
        wid = lax.axis_index("s") * 2 + lax.axis_index("c")
        s_base = wid * S_PER_W

        def add_body(i, _):
            sl = pl.ds(i * LANES, LANES)
            o_buf[sl] = x_buf[sl] + pe_buf[sl]
            return 0

        for chunk in range(N_CHUNK):
            pe_off = (s_base + chunk * ROWS) * DIM
            pltpu.sync_copy(pe_hbm.at[pl.ds(pe_off, CHUNK)], pe_buf)
            for b in range(BATCH):
                x_off = b * SEQ * DIM + pe_off
                pltpu.sync_copy(x_hbm.at[pl.ds(x_off, CHUNK)], x_buf)
                lax.fori_loop(0, CHUNK // LANES, add_body, 0)
                pltpu.sync_copy(o_buf, out_hbm.at[pl.ds(x_off, CHUNK)])

    return sc_add


_SC_ADD = _make_sc_kernel()


def kernel(x, pe_table):
    out_flat = _SC_ADD(x.reshape(-1), pe_table.reshape(-1))
    return out_flat.reshape(x.shape)


# speedup vs baseline: 1.3883x; 1.3883x over previous
import functools

import jax
import jax.numpy as jnp
from jax import lax
from jax.experimental import pallas as pl
from jax.experimental.pallas import tpu as pltpu
from jax.experimental.pallas import tpu_sc as plsc

BATCH = 4
SEQ = 2048
DIM = 2048
NW = 32
S_PER_W = SEQ // NW
ROWS = 8
CHUNK = ROWS * DIM
N_CHUNK = S_PER_W // ROWS
LANES = 16
N_IT = N_CHUNK * BATCH


def _make_sc_kernel():
    mesh = plsc.VectorSubcoreMesh(core_axis_name="c", subcore_axis_name="s")

    @functools.partial(
        pl.kernel,
        mesh=mesh,
        out_type=jax.ShapeDtypeStruct((BATCH * SEQ * DIM,), jnp.float32),
        scratch_types=[
            pltpu.VMEM((2, CHUNK), jnp.float32),
            pltpu.VMEM((2, CHUNK), jnp.float32),
            pltpu.VMEM((CHUNK,), jnp.float32),
            pltpu.SemaphoreType.DMA,
            pltpu.SemaphoreType.DMA,
            pltpu.SemaphoreType.DMA,
            pltpu.SemaphoreType.DMA,
        ],
    )
    def sc_add(x_hbm, pe_hbm, out_hbm, x_buf, o_buf, pe_buf, xs0, xs1, os0, os1):
        wid = lax.axis_index("s") * 2 + lax.axis_index("c")
        s_base = wid * S_PER_W
        x_sems = (xs0, xs1)
        o_sems = (os0, os1)

        def x_off(it):
            chunk, b = it // BATCH, it % BATCH
            return b * SEQ * DIM + (s_base + chunk * ROWS) * DIM

        x_descs = {}
        o_descs = {}
        x_descs[0] = pltpu.async_copy(
            x_hbm.at[pl.ds(x_off(0), CHUNK)], x_buf.at[0], x_sems[0])

        for it in range(N_IT):
            cur = it % 2
            chunk, b = it // BATCH, it % BATCH
            if b == 0:
                pltpu.sync_copy(
                    pe_hbm.at[pl.ds((s_base + chunk * ROWS) * DIM, CHUNK)],
                    pe_buf)
            if it + 1 < N_IT:
                x_descs[it + 1] = pltpu.async_copy(
                    x_hbm.at[pl.ds(x_off(it + 1), CHUNK)],
                    x_buf.at[(it + 1) % 2], x_sems[(it + 1) % 2])
            x_descs[it].wait()
            if it - 2 >= 0:
                o_descs[it - 2].wait()

            @plsc.parallel_loop(0, CHUNK // LANES, 1, unroll=8)
            def add_body(i):
                sl = pl.ds(i * LANES, LANES)
                o_buf[cur, sl] = x_buf[cur, sl] + pe_buf[sl]

            o_descs[it] = pltpu.async_copy(
                o_buf.at[cur], out_hbm.at[pl.ds(x_off(it), CHUNK)],
                o_sems[cur])

        o_descs[N_IT - 2].wait()
        o_descs[N_IT - 1].wait()

    return sc_add


_SC_ADD = _make_sc_kernel()


def kernel(x, pe_table):
    out_flat = _SC_ADD(x.reshape(-1), pe_table.reshape(-1))
    return out_flat.reshape(x.shape)


# speedup vs baseline: 7.3758x; 5.3129x over previous
import jax
import jax.numpy as jnp
from jax.experimental import pallas as pl

S_BLK = 1024


def _add_kernel(x_ref, pe_ref, o_ref):
    o_ref[...] = x_ref[...] + pe_ref[...]


def kernel(x, pe_table):
    batch, seq_len, embed_dim = x.shape
    n_s = seq_len // S_BLK
    return pl.pallas_call(
        _add_kernel,
        grid=(n_s, batch),
        in_specs=[
            pl.BlockSpec((1, S_BLK, embed_dim), lambda s, b: (b, s, 0)),
            pl.BlockSpec((S_BLK, embed_dim), lambda s, b: (s, 0)),
        ],
        out_specs=pl.BlockSpec((1, S_BLK, embed_dim), lambda s, b: (b, s, 0)),
        out_shape=jax.ShapeDtypeStruct(x.shape, x.dtype),
    )(x, pe_table)
